# baseline (speedup 1.0000x reference)
import functools

import jax
import jax.numpy as jnp
from jax import lax
from jax.experimental import pallas as pl
from jax.experimental.pallas import tpu as pltpu
from jax.experimental.pallas import tpu_sc as plsc

TOPK = 8
NW = 32
L = 16
NEG = -3.0e38


def _matmul_body(x_ref, wt_ref, logits_ref, ltr_ref, imp_ref, *, grid):
    pid = pl.program_id(0)
    logits = jnp.dot(x_ref[...], wt_ref[...], preferred_element_type=jnp.float32)
    logits_ref[...] = logits
    ltr_ref[...] = logits.T[None]
    m = jnp.max(logits, axis=1, keepdims=True)
    ex = jnp.exp(logits - m)
    s = jnp.sum(ex, axis=1, keepdims=True)
    probs = ex / s

    @pl.when(pid == 0)
    def _init():
        imp_ref[...] = jnp.zeros_like(imp_ref)

    imp_ref[...] += jnp.sum(probs, axis=0, keepdims=True)


def _tc_logits(hidden_states, gate_weight_t, bt):
    n_tokens, d_model = hidden_states.shape
    e_dim = gate_weight_t.shape[1]
    grid = n_tokens // bt
    body = lambda *refs: _matmul_body(*refs, grid=grid)
    return pl.pallas_call(
        body,
        grid=(grid,),
        in_specs=[
            pl.BlockSpec((bt, d_model), lambda i: (i, 0)),
            pl.BlockSpec((d_model, e_dim), lambda i: (0, 0)),
        ],
        out_specs=[
            pl.BlockSpec((bt, e_dim), lambda i: (i, 0)),
            pl.BlockSpec((1, e_dim, bt), lambda i: (i, 0, 0)),
            pl.BlockSpec((1, e_dim), lambda i: (0, 0)),
        ],
        out_shape=[
            jax.ShapeDtypeStruct((n_tokens, e_dim), jnp.float32),
            jax.ShapeDtypeStruct((grid, e_dim, bt), jnp.float32),
            jax.ShapeDtypeStruct((1, e_dim), jnp.float32),
        ],
        compiler_params=pltpu.CompilerParams(
            dimension_semantics=("arbitrary",),
        ),
    )(hidden_states, gate_weight_t)


def _make_sc_router(n_tokens, e_dim):
    tpt = n_tokens // NW
    n_groups = tpt // L

    mesh = plsc.VectorSubcoreMesh(core_axis_name="c", subcore_axis_name="s")

    @functools.partial(
        pl.kernel,
        out_type=[
            jax.ShapeDtypeStruct((TOPK, NW, tpt), jnp.int32),
            jax.ShapeDtypeStruct((TOPK, NW, tpt), jnp.float32),
            jax.ShapeDtypeStruct((NW, e_dim * L), jnp.float32),
        ],
        mesh=mesh,
        scratch_types=[
            pltpu.VMEM((e_dim * tpt,), jnp.float32),
            pltpu.VMEM((TOPK * tpt,), jnp.int32),
            pltpu.VMEM((TOPK * tpt,), jnp.float32),
            pltpu.VMEM((e_dim * L,), jnp.float32),
            pltpu.SemaphoreType.DMA,
        ],
    )
    def sc_router(lt_hbm, idx_hbm, w_hbm, cnt_hbm, lbuf, ibuf, wbuf, cntbuf,
                  sem):
        wid = lax.axis_index("s") * 2 + lax.axis_index("c")
        zeros16 = jnp.zeros((L,), jnp.float32)

        copies = [
            pltpu.async_copy(lt_hbm.at[wid, e], lbuf.at[pl.ds(e * tpt, tpt)],
                             sem)
            for e in range(e_dim)
        ]
        for cp in copies:
            cp.wait()
        for i in range(e_dim):
            cntbuf[pl.ds(i * L, L)] = zeros16

        def group_body(g, carry):
            off = g * L
            topv = [jnp.full((L,), NEG, jnp.float32)] * TOPK
            topi = [jnp.zeros((L,), jnp.int32)] * TOPK
            for e in range(e_dim):
                l = lbuf[pl.ds(e * tpt + off, L)]
                il = jnp.full((L,), e, jnp.int32)
                for j in range(TOPK):
                    c = l > topv[j]
                    nv = jnp.maximum(topv[j], l)
                    nl = jnp.minimum(topv[j], l)
                    ni = jnp.where(c, il, topi[j])
                    nil = jnp.where(c, topi[j], il)
                    topv[j] = nv
                    topi[j] = ni
                    l = nl
                    il = nil
            m = topv[0]
            exs = [jnp.exp(topv[j] - m) for j in range(TOPK)]
            ssum = exs[0]
            for j in range(1, TOPK):
                ssum = ssum + exs[j]
            rw = 1.0 / ssum
            for j in range(TOPK):
                ibuf[pl.ds(j * tpt + off, L)] = topi[j]
                wbuf[pl.ds(j * tpt + off, L)] = exs[j] * rw
            t8 = topv[TOPK - 1]
            for e in range(e_dim):
                v = lbuf[pl.ds(e * tpt + off, L)]
                hit = jnp.where(v >= t8, 1.0, 0.0)
                cntbuf[pl.ds(e * L, L)] = cntbuf[pl.ds(e * L, L)] + hit
            return carry

        lax.fori_loop(0, n_groups, group_body, 0)

        for j in range(TOPK):
            pltpu.sync_copy(ibuf.at[pl.ds(j * tpt, tpt)], idx_hbm.at[j, wid])
            pltpu.sync_copy(wbuf.at[pl.ds(j * tpt, tpt)], w_hbm.at[j, wid])
        pltpu.sync_copy(cntbuf, cnt_hbm.at[wid])

    return sc_router


def _loss_body(imp_ref, loadp_ref, loss_ref, *, n_tokens, e_dim):
    load = jnp.sum(loadp_ref[...], axis=0, keepdims=True)
    scale = jnp.float32(e_dim) / (jnp.float32(n_tokens) *
                                  jnp.float32(n_tokens * TOPK))
    loss_ref[...] = scale * jnp.sum(imp_ref[...] * load, axis=(0, 1),
                                    keepdims=True)


def _tc_loss(imp, loadp, n_tokens, e_dim):
    body = lambda *refs: _loss_body(*refs, n_tokens=n_tokens, e_dim=e_dim)
    return pl.pallas_call(
        body,
        out_shape=jax.ShapeDtypeStruct((1, 1), jnp.float32),
    )(imp, loadp)


def kernel(hidden_states, gate_weight):
    n_tokens, d_model = hidden_states.shape
    e_dim = gate_weight.shape[0]
    tpt = n_tokens // NW
    logits, logits_t, imp = _tc_logits(hidden_states, gate_weight.T, tpt)
    idx_t, w_t, cnt = _make_sc_router(n_tokens, e_dim)(logits_t)
    idxs = idx_t.reshape(TOPK, n_tokens).T
    weights = w_t.reshape(TOPK, n_tokens).T
    loadp = cnt.reshape(NW, e_dim, L).transpose(0, 2, 1).reshape(NW * L, e_dim)
    loss = _tc_loss(imp, loadp, n_tokens, e_dim)
    return idxs, weights, logits, loss[0, 0]


# speedup vs baseline: 1.0436x; 1.0436x over previous
import functools

import jax
import jax.numpy as jnp
from jax import lax
from jax.experimental import pallas as pl
from jax.experimental.pallas import tpu as pltpu
from jax.experimental.pallas import tpu_sc as plsc

TOPK = 8
NW = 32
L = 16
NEG = -3.0e38

SORT8 = [(0, 1), (2, 3), (4, 5), (6, 7),
         (0, 2), (1, 3), (4, 6), (5, 7),
         (1, 2), (5, 6),
         (0, 4), (1, 5), (2, 6), (3, 7),
         (2, 4), (3, 5),
         (1, 2), (3, 4), (5, 6)]
BITONIC8 = [(0, 4), (1, 5), (2, 6), (3, 7),
            (0, 2), (1, 3), (4, 6), (5, 7),
            (0, 1), (2, 3), (4, 5), (6, 7)]


def _ce_desc(v, i, a, b):
    c = v[a] >= v[b]
    hv = jnp.maximum(v[a], v[b])
    lv = jnp.minimum(v[a], v[b])
    hi = jnp.where(c, i[a], i[b])
    li = jnp.where(c, i[b], i[a])
    v[a], v[b], i[a], i[b] = hv, lv, hi, li


def _matmul_body(x_ref, wt_ref, logits_ref, ltr_ref, imp_ref, *, grid):
    pid = pl.program_id(0)
    logits = jnp.dot(x_ref[...], wt_ref[...], preferred_element_type=jnp.float32)
    logits_ref[...] = logits
    ltr_ref[...] = logits.T[None]
    m = jnp.max(logits, axis=1, keepdims=True)
    ex = jnp.exp(logits - m)
    s = jnp.sum(ex, axis=1, keepdims=True)
    probs = ex / s

    @pl.when(pid == 0)
    def _init():
        imp_ref[...] = jnp.zeros_like(imp_ref)

    imp_ref[...] += jnp.sum(probs, axis=0, keepdims=True)


def _tc_logits(hidden_states, gate_weight_t, bt):
    n_tokens, d_model = hidden_states.shape
    e_dim = gate_weight_t.shape[1]
    grid = n_tokens // bt
    body = lambda *refs: _matmul_body(*refs, grid=grid)
    return pl.pallas_call(
        body,
        grid=(grid,),
        in_specs=[
            pl.BlockSpec((bt, d_model), lambda i: (i, 0)),
            pl.BlockSpec((d_model, e_dim), lambda i: (0, 0)),
        ],
        out_specs=[
            pl.BlockSpec((bt, e_dim), lambda i: (i, 0)),
            pl.BlockSpec((1, e_dim, bt), lambda i: (i, 0, 0)),
            pl.BlockSpec((1, e_dim), lambda i: (0, 0)),
        ],
        out_shape=[
            jax.ShapeDtypeStruct((n_tokens, e_dim), jnp.float32),
            jax.ShapeDtypeStruct((grid, e_dim, bt), jnp.float32),
            jax.ShapeDtypeStruct((1, e_dim), jnp.float32),
        ],
        compiler_params=pltpu.CompilerParams(
            dimension_semantics=("arbitrary",),
        ),
    )(hidden_states, gate_weight_t)


def _make_sc_router(n_tokens, e_dim):
    tpt = n_tokens // NW
    n_groups = tpt // L

    mesh = plsc.VectorSubcoreMesh(core_axis_name="c", subcore_axis_name="s")

    @functools.partial(
        pl.kernel,
        out_type=[
            jax.ShapeDtypeStruct((TOPK, NW, tpt), jnp.int32),
            jax.ShapeDtypeStruct((TOPK, NW, tpt), jnp.float32),
            jax.ShapeDtypeStruct((NW, e_dim * L), jnp.float32),
        ],
        mesh=mesh,
        scratch_types=[
            pltpu.VMEM((e_dim * tpt,), jnp.float32),
            pltpu.VMEM((TOPK * tpt,), jnp.int32),
            pltpu.VMEM((TOPK * tpt,), jnp.float32),
            pltpu.VMEM((e_dim * L,), jnp.float32),
            pltpu.SemaphoreType.DMA,
        ],
    )
    def sc_router(lt_hbm, idx_hbm, w_hbm, cnt_hbm, lbuf, ibuf, wbuf, cntbuf,
                  sem):
        wid = lax.axis_index("s") * 2 + lax.axis_index("c")
        zeros16 = jnp.zeros((L,), jnp.float32)

        copies = [
            pltpu.async_copy(lt_hbm.at[wid, e], lbuf.at[pl.ds(e * tpt, tpt)],
                             sem)
            for e in range(e_dim)
        ]
        for cp in copies:
            cp.wait()
        for i in range(e_dim):
            cntbuf[pl.ds(i * L, L)] = zeros16

        def group_body(g, carry):
            off = g * L
            topv = topi = None
            for leaf in range(e_dim // TOPK):
                v = [lbuf[pl.ds((leaf * TOPK + r) * tpt + off, L)]
                     for r in range(TOPK)]
                i = [jnp.full((L,), leaf * TOPK + r, jnp.int32)
                     for r in range(TOPK)]
                for a, b in SORT8:
                    _ce_desc(v, i, a, b)
                if topv is None:
                    topv, topi = v, i
                else:
                    cv, ci = [], []
                    for k in range(TOPK):
                        c = topv[k] >= v[TOPK - 1 - k]
                        cv.append(jnp.where(c, topv[k], v[TOPK - 1 - k]))
                        ci.append(jnp.where(c, topi[k], i[TOPK - 1 - k]))
                    for a, b in BITONIC8:
                        _ce_desc(cv, ci, a, b)
                    topv, topi = cv, ci
            m = topv[0]
            exs = [jnp.exp(topv[j] - m) for j in range(TOPK)]
            ssum = exs[0]
            for j in range(1, TOPK):
                ssum = ssum + exs[j]
            rw = 1.0 / ssum
            for j in range(TOPK):
                ibuf[pl.ds(j * tpt + off, L)] = topi[j]
                wbuf[pl.ds(j * tpt + off, L)] = exs[j] * rw
            t8 = topv[TOPK - 1]
            for e in range(e_dim):
                v = lbuf[pl.ds(e * tpt + off, L)]
                hit = jnp.where(v >= t8, 1.0, 0.0)
                cntbuf[pl.ds(e * L, L)] = cntbuf[pl.ds(e * L, L)] + hit
            return carry

        lax.fori_loop(0, n_groups, group_body, 0)

        for j in range(TOPK):
            pltpu.sync_copy(ibuf.at[pl.ds(j * tpt, tpt)], idx_hbm.at[j, wid])
            pltpu.sync_copy(wbuf.at[pl.ds(j * tpt, tpt)], w_hbm.at[j, wid])
        pltpu.sync_copy(cntbuf, cnt_hbm.at[wid])

    return sc_router


def _loss_body(imp_ref, loadp_ref, loss_ref, *, n_tokens, e_dim):
    load = jnp.sum(loadp_ref[...], axis=0, keepdims=True)
    scale = jnp.float32(e_dim) / (jnp.float32(n_tokens) *
                                  jnp.float32(n_tokens * TOPK))
    loss_ref[...] = scale * jnp.sum(imp_ref[...] * load, axis=(0, 1),
                                    keepdims=True)


def _tc_loss(imp, loadp, n_tokens, e_dim):
    body = lambda *refs: _loss_body(*refs, n_tokens=n_tokens, e_dim=e_dim)
    return pl.pallas_call(
        body,
        out_shape=jax.ShapeDtypeStruct((1, 1), jnp.float32),
    )(imp, loadp)


def kernel(hidden_states, gate_weight):
    n_tokens, d_model = hidden_states.shape
    e_dim = gate_weight.shape[0]
    tpt = n_tokens // NW
    logits, logits_t, imp = _tc_logits(hidden_states, gate_weight.T, tpt)
    idx_t, w_t, cnt = _make_sc_router(n_tokens, e_dim)(logits_t)
    idxs = idx_t.reshape(TOPK, n_tokens).T
    weights = w_t.reshape(TOPK, n_tokens).T
    loadp = cnt.reshape(NW, e_dim, L).transpose(0, 2, 1).reshape(NW * L, e_dim)
    loss = _tc_loss(imp, loadp, n_tokens, e_dim)
    return idxs, weights, logits, loss[0, 0]


# speedup vs baseline: 1.0489x; 1.0051x over previous
import functools

import jax
import jax.numpy as jnp
from jax import lax
from jax.experimental import pallas as pl
from jax.experimental.pallas import tpu as pltpu
from jax.experimental.pallas import tpu_sc as plsc

TOPK = 8
NW = 32
L = 16

SORT8 = [(0, 1), (2, 3), (4, 5), (6, 7),
         (0, 2), (1, 3), (4, 6), (5, 7),
         (1, 2), (5, 6),
         (0, 4), (1, 5), (2, 6), (3, 7),
         (2, 4), (3, 5),
         (1, 2), (3, 4), (5, 6)]
BITONIC8 = [(0, 4), (1, 5), (2, 6), (3, 7),
            (0, 2), (1, 3), (4, 6), (5, 7),
            (0, 1), (2, 3), (4, 5), (6, 7)]


def _ce_desc(v, i, a, b):
    c = v[a] >= v[b]
    hv = jnp.maximum(v[a], v[b])
    lv = jnp.minimum(v[a], v[b])
    hi = jnp.where(c, i[a], i[b])
    li = jnp.where(c, i[b], i[a])
    v[a], v[b], i[a], i[b] = hv, lv, hi, li


def _matmul_body(x_ref, wt_ref, logits_ref, ltr_ref, imp_ref, *, grid):
    pid = pl.program_id(0)
    logits = jnp.dot(x_ref[...], wt_ref[...], preferred_element_type=jnp.float32)
    logits_ref[...] = logits
    ltr_ref[...] = logits.T[None]
    m = jnp.max(logits, axis=1, keepdims=True)
    ex = jnp.exp(logits - m)
    s = jnp.sum(ex, axis=1, keepdims=True)
    probs = ex / s

    @pl.when(pid == 0)
    def _init():
        imp_ref[...] = jnp.zeros_like(imp_ref)

    imp_ref[...] += jnp.sum(probs, axis=0, keepdims=True)


def _tc_logits(hidden_states, gate_weight_t, bt):
    n_tokens, d_model = hidden_states.shape
    e_dim = gate_weight_t.shape[1]
    grid = n_tokens // bt
    body = lambda *refs: _matmul_body(*refs, grid=grid)
    return pl.pallas_call(
        body,
        grid=(grid,),
        in_specs=[
            pl.BlockSpec((bt, d_model), lambda i: (i, 0)),
            pl.BlockSpec((d_model, e_dim), lambda i: (0, 0)),
        ],
        out_specs=[
            pl.BlockSpec((bt, e_dim), lambda i: (i, 0)),
            pl.BlockSpec((1, e_dim, bt), lambda i: (i, 0, 0)),
            pl.BlockSpec((1, e_dim), lambda i: (0, 0)),
        ],
        out_shape=[
            jax.ShapeDtypeStruct((n_tokens, e_dim), jnp.float32),
            jax.ShapeDtypeStruct((grid, e_dim, bt), jnp.float32),
            jax.ShapeDtypeStruct((1, e_dim), jnp.float32),
        ],
        compiler_params=pltpu.CompilerParams(
            dimension_semantics=("arbitrary",),
        ),
    )(hidden_states, gate_weight_t)


def _make_sc_router(n_tokens, e_dim):
    tpt = n_tokens // NW
    n_groups = tpt // L

    mesh = plsc.VectorSubcoreMesh(core_axis_name="c", subcore_axis_name="s")

    @functools.partial(
        pl.kernel,
        out_type=[
            jax.ShapeDtypeStruct((TOPK, NW, tpt), jnp.int32),
            jax.ShapeDtypeStruct((TOPK, NW, tpt), jnp.float32),
            jax.ShapeDtypeStruct((NW, e_dim * L), jnp.float32),
        ],
        mesh=mesh,
        scratch_types=[
            pltpu.VMEM((e_dim * tpt,), jnp.float32),
            pltpu.VMEM((TOPK * tpt,), jnp.int32),
            pltpu.VMEM((TOPK * tpt,), jnp.float32),
            pltpu.VMEM((e_dim * L,), jnp.float32),
            pltpu.SemaphoreType.DMA,
        ],
    )
    def sc_router(lt_hbm, idx_hbm, w_hbm, cnt_hbm, lbuf, ibuf, wbuf, cntbuf,
                  sem):
        wid = lax.axis_index("s") * 2 + lax.axis_index("c")
        zeros16 = jnp.zeros((L,), jnp.float32)

        copies = [
            pltpu.async_copy(lt_hbm.at[wid, e], lbuf.at[pl.ds(e * tpt, tpt)],
                             sem)
            for e in range(e_dim)
        ]
        for cp in copies:
            cp.wait()
        for i in range(e_dim):
            cntbuf[pl.ds(i * L, L)] = zeros16

        def group_body(g, carry):
            off = g * L
            topv = topi = None
            for leaf in range(e_dim // TOPK):
                v = [lbuf[pl.ds((leaf * TOPK + r) * tpt + off, L)]
                     for r in range(TOPK)]
                i = [jnp.full((L,), leaf * TOPK + r, jnp.int32)
                     for r in range(TOPK)]
                for a, b in SORT8:
                    _ce_desc(v, i, a, b)
                if topv is None:
                    topv, topi = v, i
                else:
                    cv, ci = [], []
                    for k in range(TOPK):
                        c = topv[k] >= v[TOPK - 1 - k]
                        cv.append(jnp.where(c, topv[k], v[TOPK - 1 - k]))
                        ci.append(jnp.where(c, topi[k], i[TOPK - 1 - k]))
                    for a, b in BITONIC8:
                        _ce_desc(cv, ci, a, b)
                    topv, topi = cv, ci
            m = topv[0]
            exs = [jnp.exp(topv[j] - m) for j in range(TOPK)]
            ssum = exs[0]
            for j in range(1, TOPK):
                ssum = ssum + exs[j]
            rw = 1.0 / ssum
            for j in range(TOPK):
                ibuf[pl.ds(j * tpt + off, L)] = topi[j]
                wbuf[pl.ds(j * tpt + off, L)] = exs[j] * rw
            t8 = topv[TOPK - 1]
            for e in range(e_dim):
                v = lbuf[pl.ds(e * tpt + off, L)]
                hit = jnp.where(v >= t8, 1.0, 0.0)
                cntbuf[pl.ds(e * L, L)] = cntbuf[pl.ds(e * L, L)] + hit
            return carry

        lax.fori_loop(0, n_groups, group_body, 0)

        for j in range(TOPK):
            pltpu.sync_copy(ibuf.at[pl.ds(j * tpt, tpt)], idx_hbm.at[j, wid])
            pltpu.sync_copy(wbuf.at[pl.ds(j * tpt, tpt)], w_hbm.at[j, wid])
        pltpu.sync_copy(cntbuf, cnt_hbm.at[wid])

    return sc_router


def _loss_body(imp_ref, loadp_ref, loss_ref, *, n_tokens, e_dim):
    imp = jnp.sum(imp_ref[...], axis=0, keepdims=True)
    load = jnp.sum(loadp_ref[...], axis=0, keepdims=True)
    scale = jnp.float32(e_dim) / (jnp.float32(n_tokens) *
                                  jnp.float32(n_tokens * TOPK))
    loss_ref[...] = scale * jnp.sum(imp * load, axis=(0, 1), keepdims=True)


def _tc_loss(imp, loadp, n_tokens, e_dim):
    body = lambda *refs: _loss_body(*refs, n_tokens=n_tokens, e_dim=e_dim)
    return pl.pallas_call(
        body,
        out_shape=jax.ShapeDtypeStruct((1, 1), jnp.float32),
    )(imp, loadp)


def kernel(hidden_states, gate_weight):
    n_tokens, d_model = hidden_states.shape
    e_dim = gate_weight.shape[0]
    tpt = n_tokens // NW
    logits, logits_t, imp = _tc_logits(hidden_states, gate_weight.T, tpt)
    idx_t, w_t, cnt = _make_sc_router(n_tokens, e_dim)(logits_t)
    idxs = idx_t.reshape(TOPK, n_tokens).T
    weights = w_t.reshape(TOPK, n_tokens).T
    loadp = cnt.reshape(NW, e_dim, L).transpose(0, 2, 1).reshape(NW * L, e_dim)
    loss = _tc_loss(imp, loadp, n_tokens, e_dim)
    return idxs, weights, logits, loss[0, 0]
